# baseline (speedup 1.0000x reference)
import dataclasses

import jax
import jax.numpy as jnp
from jax import lax
from jax.experimental import pallas as pl
from jax.experimental.pallas import tpu as pltpu
from jax.experimental.pallas import tpu_sc as plsc

B = 16384
EMB = 32
L = 16
NC, NS = 2, 16
NW = NC * NS
BPW = B // NW


def _mf_body(ux_hbm, ix_hbm, u_hbm, v_hbm, out_hbm,
             uidx_v, vidx_v, urows_v, vrows_v, out_v, sem_u, sem_v):
    wid = lax.axis_index("s") * NC + lax.axis_index("c")
    base = wid * BPW
    pltpu.sync_copy(ux_hbm.at[pl.ds(base, BPW)], uidx_v)
    pltpu.sync_copy(ix_hbm.at[pl.ds(base, BPW)], vidx_v)
    cu = pltpu.async_copy(u_hbm.at[uidx_v], urows_v, sem_u)
    cv = pltpu.async_copy(v_hbm.at[vidx_v], vrows_v, sem_v)
    cu.wait()
    cv.wait()

    last_lane = lax.iota(jnp.int32, L) == (L - 1)

    @pl.loop(0, BPW)
    def _(r):
        p = (urows_v[r, pl.ds(0, L)] * vrows_v[r, pl.ds(0, L)]
             + urows_v[r, pl.ds(L, L)] * vrows_v[r, pl.ds(L, L)])
        s = plsc.cumsum(p)
        plsc.store_compressed(out_v.at[pl.ds(r, L)], s, mask=last_lane)

    pltpu.sync_copy(out_v.at[pl.ds(0, BPW)], out_hbm.at[pl.ds(base, BPW)])


def kernel(ux, ix, U, V):
    mesh = plsc.VectorSubcoreMesh(core_axis_name="c", subcore_axis_name="s")
    cp = pltpu.CompilerParams()
    if "needs_layout_passes" in pltpu.CompilerParams.__dataclass_fields__:
        cp = dataclasses.replace(cp, needs_layout_passes=False)
    if "use_tc_tiling_on_sc" in pltpu.CompilerParams.__dataclass_fields__:
        cp = dataclasses.replace(cp, use_tc_tiling_on_sc=False)
    k = pl.kernel(
        _mf_body,
        out_type=jax.ShapeDtypeStruct((B,), jnp.float32),
        mesh=mesh,
        scratch_types=[
            pltpu.VMEM((BPW,), jnp.int32),
            pltpu.VMEM((BPW,), jnp.int32),
            pltpu.VMEM((BPW, EMB), jnp.float32),
            pltpu.VMEM((BPW, EMB), jnp.float32),
            pltpu.VMEM((BPW + L,), jnp.float32),
            pltpu.SemaphoreType.DMA,
            pltpu.SemaphoreType.DMA,
        ],
        compiler_params=cp,
    )
    return k(ux.astype(jnp.int32), ix.astype(jnp.int32), U, V)


# speedup vs baseline: 4.4617x; 4.4617x over previous
import dataclasses

import jax
import jax.numpy as jnp
from jax import lax
from jax.experimental import pallas as pl
from jax.experimental.pallas import tpu as pltpu
from jax.experimental.pallas import tpu_sc as plsc

B = 16384
EMB = 32
L = 16
NC, NS = 2, 16
NW = NC * NS
BPW = B // NW
RING = 8
LANE = 128


def _win_off(i):
    return pl.multiple_of((i >> 7) * LANE, LANE)


def _mf_body(ux_hbm, ix_hbm, ut_hbm, vt_hbm, out_hbm,
             uidx_v, vidx_v, ublk, vblk, out_v, usem, vsem):
    wid = lax.axis_index("s") * NC + lax.axis_index("c")
    base = wid * BPW
    pltpu.sync_copy(ux_hbm.at[pl.ds(base, BPW)], uidx_v.at[pl.ds(0, BPW)])
    pltpu.sync_copy(ix_hbm.at[pl.ds(base, BPW)], vidx_v.at[pl.ds(0, BPW)])

    def idx_at(ref, e):
        return ref[pl.ds(e, L)][0]

    def issue(j, e):
        ui = idx_at(uidx_v, e)
        vi = idx_at(vidx_v, e)
        pltpu.async_copy(ut_hbm.at[:, pl.ds(_win_off(ui), LANE)], ublk[j], usem)
        pltpu.async_copy(vt_hbm.at[:, pl.ds(_win_off(vi), LANE)], vblk[j], vsem)

    for j in range(RING):
        issue(j, j)

    c_lo = lax.iota(jnp.int32, L)
    c_hi = c_lo + L
    last_lane = c_lo == (L - 1)

    @pl.loop(0, BPW // RING)
    def _(g):
        e0 = g * RING
        for j in range(RING):
            e = e0 + j
            pltpu.make_async_copy(
                ut_hbm.at[:, pl.ds(0, LANE)], ublk[j], usem).wait()
            pltpu.make_async_copy(
                vt_hbm.at[:, pl.ds(0, LANE)], vblk[j], vsem).wait()
            lu = jnp.full((L,), idx_at(uidx_v, e) & (LANE - 1), jnp.int32)
            lv = jnp.full((L,), idx_at(vidx_v, e) & (LANE - 1), jnp.int32)
            u0 = plsc.load_gather(ublk[j], [c_lo, lu])
            u1 = plsc.load_gather(ublk[j], [c_hi, lu])
            v0 = plsc.load_gather(vblk[j], [c_lo, lv])
            v1 = plsc.load_gather(vblk[j], [c_hi, lv])
            s = plsc.cumsum(u0 * v0 + u1 * v1)
            plsc.store_compressed(out_v.at[pl.ds(e, L)], s, mask=last_lane)

            @pl.when(e + RING < BPW)
            def _():
                issue(j, e + RING)

    pltpu.sync_copy(out_v.at[pl.ds(0, BPW)], out_hbm.at[pl.ds(base, BPW)])


def kernel(ux, ix, U, V):
    mesh = plsc.VectorSubcoreMesh(core_axis_name="c", subcore_axis_name="s")
    cp = pltpu.CompilerParams()
    if "needs_layout_passes" in pltpu.CompilerParams.__dataclass_fields__:
        cp = dataclasses.replace(cp, needs_layout_passes=False)
    if "use_tc_tiling_on_sc" in pltpu.CompilerParams.__dataclass_fields__:
        cp = dataclasses.replace(cp, use_tc_tiling_on_sc=True)
    k = pl.kernel(
        _mf_body,
        out_type=jax.ShapeDtypeStruct((B,), jnp.float32),
        mesh=mesh,
        scratch_types=[
            pltpu.VMEM((BPW + L,), jnp.int32),
            pltpu.VMEM((BPW + L,), jnp.int32),
            [pltpu.VMEM((EMB, LANE), jnp.float32) for _ in range(RING)],
            [pltpu.VMEM((EMB, LANE), jnp.float32) for _ in range(RING)],
            pltpu.VMEM((BPW + L,), jnp.float32),
            pltpu.SemaphoreType.DMA,
            pltpu.SemaphoreType.DMA,
        ],
        compiler_params=cp,
    )
    return k(ux.astype(jnp.int32), ix.astype(jnp.int32), U.T, V.T)


# speedup vs baseline: 4.4992x; 1.0084x over previous
import dataclasses

import jax
import jax.numpy as jnp
from jax import lax
from jax.experimental import pallas as pl
from jax.experimental.pallas import tpu as pltpu
from jax.experimental.pallas import tpu_sc as plsc

B = 16384
EMB = 32
L = 16
NC, NS = 2, 16
NW = NC * NS
BPW = B // NW
RING = 8
LANE = 128


def _win_off(i):
    return pl.multiple_of((i >> 7) * LANE, LANE)


def _mf_body(ux_hbm, ix_hbm, ut_hbm, vt_hbm, out_hbm,
             uidx_v, vidx_v, ublk, vblk, out_v, usem, vsem):
    wid = lax.axis_index("s") * NC + lax.axis_index("c")
    base = wid * BPW
    pltpu.sync_copy(ux_hbm.at[pl.ds(base, BPW)], uidx_v.at[pl.ds(0, BPW)])
    pltpu.sync_copy(ix_hbm.at[pl.ds(base, BPW)], vidx_v.at[pl.ds(0, BPW)])

    def issue(j, ui, vi):
        pltpu.async_copy(ut_hbm.at[:, pl.ds(_win_off(ui), LANE)], ublk[j], usem)
        pltpu.async_copy(vt_hbm.at[:, pl.ds(_win_off(vi), LANE)], vblk[j], vsem)

    pvec_u = uidx_v[pl.ds(0, L)]
    pvec_v = vidx_v[pl.ds(0, L)]
    for j in range(RING):
        issue(j, pvec_u[j], pvec_v[j])

    c_lo = lax.iota(jnp.int32, L)
    c_hi = c_lo + L
    last_lane = c_lo == (L - 1)

    @pl.loop(0, BPW // RING)
    def _(g):
        e0 = g * RING
        uvec = uidx_v[pl.ds(e0, L)]
        vvec = vidx_v[pl.ds(e0, L)]
        lu_all = uvec & (LANE - 1)
        lv_all = vvec & (LANE - 1)
        for j in range(RING):
            e = e0 + j
            pltpu.make_async_copy(
                ut_hbm.at[:, pl.ds(0, LANE)], ublk[j], usem).wait()
            pltpu.make_async_copy(
                vt_hbm.at[:, pl.ds(0, LANE)], vblk[j], vsem).wait()
            lu = jnp.full((L,), lu_all[j], jnp.int32)
            lv = jnp.full((L,), lv_all[j], jnp.int32)
            u0 = plsc.load_gather(ublk[j], [c_lo, lu])
            u1 = plsc.load_gather(ublk[j], [c_hi, lu])
            v0 = plsc.load_gather(vblk[j], [c_lo, lv])
            v1 = plsc.load_gather(vblk[j], [c_hi, lv])
            s = plsc.cumsum(u0 * v0 + u1 * v1)
            plsc.store_compressed(out_v.at[pl.ds(e, L)], s, mask=last_lane)

            @pl.when(e + RING < BPW)
            def _():
                issue(j, uvec[j + RING], vvec[j + RING])

    pltpu.sync_copy(out_v.at[pl.ds(0, BPW)], out_hbm.at[pl.ds(base, BPW)])


def kernel(ux, ix, U, V):
    mesh = plsc.VectorSubcoreMesh(core_axis_name="c", subcore_axis_name="s")
    cp = pltpu.CompilerParams()
    if "needs_layout_passes" in pltpu.CompilerParams.__dataclass_fields__:
        cp = dataclasses.replace(cp, needs_layout_passes=False)
    if "use_tc_tiling_on_sc" in pltpu.CompilerParams.__dataclass_fields__:
        cp = dataclasses.replace(cp, use_tc_tiling_on_sc=True)
    k = pl.kernel(
        _mf_body,
        out_type=jax.ShapeDtypeStruct((B,), jnp.float32),
        mesh=mesh,
        scratch_types=[
            pltpu.VMEM((BPW + L,), jnp.int32),
            pltpu.VMEM((BPW + L,), jnp.int32),
            [pltpu.VMEM((EMB, LANE), jnp.float32) for _ in range(RING)],
            [pltpu.VMEM((EMB, LANE), jnp.float32) for _ in range(RING)],
            pltpu.VMEM((BPW + L,), jnp.float32),
            pltpu.SemaphoreType.DMA,
            pltpu.SemaphoreType.DMA,
        ],
        compiler_params=cp,
    )
    return k(ux.astype(jnp.int32), ix.astype(jnp.int32), U.T, V.T)
